# baseline (speedup 1.0000x reference)
import functools

import numpy as np
import jax
import jax.numpy as jnp
from jax import lax
from jax.experimental import pallas as pl
from jax.experimental.pallas import tpu as pltpu
from jax.experimental.pallas import tpu_sc as plsc

NC = 2
NS = 16
L = 16
NW = NC * NS
BINS = 10
NB = BINS + 1
COLS = 1024
CROWS = 16
CHUNK = CROWS * COLS
UNROLL = 4
TCBR = 512
ROWS1_SC = 8192
TCBR2 = 256
ROWS2_SC = 9216
THRESH10 = float(np.float32(10.0) * (np.float32(1.0) + np.float32(1e-6)))


def _wid():
    return lax.axis_index("s") * NC + lax.axis_index("c")


def _mesh():
    return plsc.VectorSubcoreMesh(
        core_axis_name="c", subcore_axis_name="s", num_cores=NC, num_subcores=NS
    )


def _stream_loop(arrays, bufs0, bufs1, sem0, sem1, nchunk, compute, carry):
    row0 = _wid() * (nchunk * CROWS)

    def start(bufs, sem, k):
        r = row0 + k * CROWS
        for a, b in zip(arrays, bufs):
            pltpu.async_copy(a.at[pl.ds(r, CROWS), :], b, sem)

    def wait(bufs, sem, k):
        r = row0 + k * CROWS
        for a, b in zip(arrays, bufs):
            pltpu.make_async_copy(a.at[pl.ds(r, CROWS), :], b, sem).wait()

    start(bufs0, sem0, 0)

    def outer(k, carry):
        start(bufs1, sem1, 2 * k + 1)
        wait(bufs0, sem0, 2 * k)
        carry = compute(bufs0, carry)
        start(bufs0, sem0, 2 * k + 2)
        wait(bufs1, sem1, 2 * k + 1)
        carry = compute(bufs1, carry)
        return carry

    carry = lax.fori_loop(0, nchunk // 2 - 1, outer, carry)
    start(bufs1, sem1, nchunk - 1)
    wait(bufs0, sem0, nchunk - 2)
    carry = compute(bufs0, carry)
    wait(bufs1, sem1, nchunk - 1)
    carry = compute(bufs1, carry)
    return carry


def _group(buf, i):
    return buf[lax.shift_right_logical(i, 10), pl.ds(lax.bitwise_and(i, COLS - 1), L)]


def _pass1_body(nchunk, p_hbm, t_hbm, out_hbm, pb0, tb0, pb1, tb1, accb, sem0, sem1):
    def compute(bufs, acc):
        pb, tb = bufs

        def inner(j, acc):
            acc = list(acc)
            for u in range(UNROLL):
                o = j * (L * UNROLL) + u * L
                p = _group(pb, o)
                t = _group(tb, o)
                a_i, a_p, a_t = acc[u]
                acc[u] = (a_i + p * t, a_p + p, a_t + t)
            return tuple(acc)

        return lax.fori_loop(0, CHUNK // (L * UNROLL), inner, acc)

    z = jnp.zeros((L,), jnp.float32)
    acc = _stream_loop((p_hbm, t_hbm), (pb0, tb0), (pb1, tb1),
                       sem0, sem1, nchunk, compute, ((z, z, z),) * UNROLL)
    for i in range(3):
        v = acc[0][i]
        for u in range(1, UNROLL):
            v = v + acc[u][i]
        accb[i, :] = v
    accb[3, :] = jnp.zeros((L,), jnp.float32)
    pltpu.sync_copy(accb, out_hbm.at[_wid()])


def _tc1_body(p_ref, t_ref, o_ref):
    @pl.when(pl.program_id(0) == 0)
    def _init():
        o_ref[...] = jnp.zeros_like(o_ref)

    p = p_ref[...]
    t = t_ref[...]
    for arr, base in ((p * t, 0), (p, 8), (t, 16)):
        s = arr[0:8, :]
        for i in range(1, TCBR // 8):
            s = s + arr[8 * i:8 * (i + 1), :]
        o_ref[base:base + 8, :] += s


def _tc2_body(c_ref, p_ref, t_ref, w_ref, o_ref):
    @pl.when(pl.program_id(0) == 0)
    def _init():
        o_ref[...] = jnp.zeros_like(o_ref)

    c10 = c_ref[0]
    edges = [float(b) for b in range(BINS)] + [THRESH10]

    p = p_ref[...]
    t = t_ref[...]
    w = w_ref[...]
    g = jnp.abs(c10 * p - 10.0 * t)
    gm = jnp.where(w > 0.0, g, -1.0)
    pt = p * t
    for e_idx, e in enumerate(edges):
        mask = gm >= e
        cnt = mask.astype(jnp.float32)
        ptm = jnp.where(mask, pt, 0.0)
        a_c = cnt[0:8, :]
        a_p = ptm[0:8, :]
        for i in range(1, TCBR2 // 8):
            a_c = a_c + cnt[8 * i:8 * (i + 1), :]
            a_p = a_p + ptm[8 * i:8 * (i + 1), :]
        o_ref[e_idx, :, :] += a_c
        o_ref[NB + e_idx, :, :] += a_p


def _pass2_body(nchunk, p_hbm, t_hbm, w_hbm, c_hbm, out_hbm,
                pb0, tb0, wb0, pb1, tb1, wb1, cb, hist, sem0, sem1):
    pltpu.sync_copy(c_hbm, cb)
    c10 = cb[...]
    for i in range(2 * NB):
        hist[pl.ds(i * L, L)] = jnp.zeros((L,), jnp.float32)
    lane = lax.iota(jnp.int32, L)
    ones = jnp.ones((L,), jnp.float32)

    def compute(bufs, carry):
        pb, tb, wb = bufs

        @plsc.parallel_loop(0, CHUNK, L, unroll=8)
        def body(i):
            p = _group(pb, i)
            t = _group(tb, i)
            w = _group(wb, i)
            g10 = jnp.abs(c10 * p - 10.0 * t)
            b = jnp.where(g10 < THRESH10,
                          jnp.minimum(g10.astype(jnp.int32), BINS - 1), BINS)
            m = w > 0.0
            slot = b * L + lane
            plsc.addupdate_scatter(hist, [slot], ones, mask=m)
            plsc.addupdate_scatter(hist, [slot + NB * L], p * t, mask=m)

        return carry

    _stream_loop((p_hbm, t_hbm, w_hbm), (pb0, tb0, wb0), (pb1, tb1, wb1),
                 sem0, sem1, nchunk, compute, 0)
    pltpu.sync_copy(hist, out_hbm.at[_wid()])


@functools.cache
def _build(nrows):
    assert nrows % (NW * CROWS * 2) == 0, nrows
    nchunk1 = ROWS1_SC // (NW * CROWS)
    nchunk2 = ROWS2_SC // (NW * CROWS)
    assert nchunk1 % 2 == 0 and nchunk2 % 2 == 0
    rows_tc = nrows - ROWS1_SC
    row_off = ROWS1_SC // TCBR
    rows_tc2 = nrows - ROWS2_SC
    row_off2 = ROWS2_SC // TCBR2
    buf = lambda: pltpu.VMEM((CROWS, COLS), jnp.float32)
    params = pltpu.CompilerParams(needs_layout_passes=False)

    pass1 = pl.kernel(
        functools.partial(_pass1_body, nchunk1),
        out_type=jax.ShapeDtypeStruct((NW, 4, L), jnp.float32),
        mesh=_mesh(),
        compiler_params=params,
        scratch_types=[buf() for _ in range(4)]
        + [pltpu.VMEM((4, L), jnp.float32),
           pltpu.SemaphoreType.DMA, pltpu.SemaphoreType.DMA],
    )
    tc1 = pl.pallas_call(
        _tc1_body,
        grid=(rows_tc // TCBR,),
        in_specs=[pl.BlockSpec((TCBR, COLS), lambda k: (row_off + k, 0)),
                  pl.BlockSpec((TCBR, COLS), lambda k: (row_off + k, 0))],
        out_specs=pl.BlockSpec((24, COLS), lambda k: (0, 0)),
        out_shape=jax.ShapeDtypeStruct((24, COLS), jnp.float32),
    )
    tc2 = pl.pallas_call(
        _tc2_body,
        grid=(rows_tc2 // TCBR2,),
        in_specs=[pl.BlockSpec(memory_space=pltpu.SMEM),
                  pl.BlockSpec((TCBR2, COLS), lambda k: (row_off2 + k, 0)),
                  pl.BlockSpec((TCBR2, COLS), lambda k: (row_off2 + k, 0)),
                  pl.BlockSpec((TCBR2, COLS), lambda k: (row_off2 + k, 0))],
        out_specs=pl.BlockSpec((2 * NB, 8, COLS), lambda k: (0, 0, 0)),
        out_shape=jax.ShapeDtypeStruct((2 * NB, 8, COLS), jnp.float32),
    )
    pass2 = pl.kernel(
        functools.partial(_pass2_body, nchunk2),
        out_type=jax.ShapeDtypeStruct((NW, 2 * NB * L), jnp.float32),
        mesh=_mesh(),
        compiler_params=params,
        scratch_types=[buf() for _ in range(6)]
        + [pltpu.VMEM((L,), jnp.float32),
           pltpu.VMEM((2 * NB * L,), jnp.float32),
           pltpu.SemaphoreType.DMA, pltpu.SemaphoreType.DMA],
    )
    return pass1, tc1, pass2, tc2


def kernel(pred, target, label_weight):
    p = pred
    t = target.astype(jnp.float32)
    w = label_weight.astype(jnp.float32)
    pass1, tc1, pass2, tc2 = _build(p.shape[0])

    part1 = pass1(p, t)
    part1_tc = tc1(p, t)
    s = jnp.sum(part1, axis=(0, 2))
    i_sum = s[0] + jnp.sum(part1_tc[0:8])
    big_s = (s[1] + s[2]
             + jnp.sum(part1_tc[8:16]) + jnp.sum(part1_tc[16:24]))
    cvec = jnp.full((L,), 20.0 * i_sum / big_s, jnp.float32)

    part2 = pass2(p, t, w, cvec)
    part2_tc = tc2(cvec[:1], p, t, w)
    h = jnp.sum(part2.reshape(NW, 2 * NB, L), axis=(0, 2))
    cum_c = jnp.sum(part2_tc[:NB], axis=(1, 2))
    cum_p = jnp.sum(part2_tc[NB:], axis=(1, 2))
    zero1 = jnp.zeros((1,), jnp.float32)
    tc_counts = cum_c - jnp.concatenate([cum_c[1:], zero1])
    tc_pt = cum_p - jnp.concatenate([cum_p[1:], zero1])
    counts = h[:BINS] + tc_counts[:BINS]
    tot = jnp.maximum(jnp.sum(h[:NB]) + cum_c[0], 1.0)
    ptb = h[NB:NB + BINS] + tc_pt[:BINS]
    n = jnp.sum((counts > 0).astype(jnp.float32))
    contrib = jnp.where(counts > 0, tot / jnp.maximum(counts, 1.0), 0.0) * ptb
    loss = 1.0 - (2.0 * jnp.sum(contrib) / jnp.maximum(n, 1.0)) / big_s
    return loss


# speedup vs baseline: 1.0278x; 1.0278x over previous
import functools

import numpy as np
import jax
import jax.numpy as jnp
from jax import lax
from jax.experimental import pallas as pl
from jax.experimental.pallas import tpu as pltpu
from jax.experimental.pallas import tpu_sc as plsc

NC = 2
NS = 16
L = 16
NW = NC * NS
BINS = 10
NB = BINS + 1
COLS = 1024
CROWS = 16
CHUNK = CROWS * COLS
UNROLL = 4
TCBR = 512
ROWS1_SC = 8192
TCBR2 = 512
ROWS2_SC = 10240
THRESH10 = float(np.float32(10.0) * (np.float32(1.0) + np.float32(1e-6)))


def _wid():
    return lax.axis_index("s") * NC + lax.axis_index("c")


def _mesh():
    return plsc.VectorSubcoreMesh(
        core_axis_name="c", subcore_axis_name="s", num_cores=NC, num_subcores=NS
    )


def _stream_loop(arrays, bufs0, bufs1, sem0, sem1, nchunk, compute, carry):
    row0 = _wid() * (nchunk * CROWS)

    def start(bufs, sem, k):
        r = row0 + k * CROWS
        for a, b in zip(arrays, bufs):
            pltpu.async_copy(a.at[pl.ds(r, CROWS), :], b, sem)

    def wait(bufs, sem, k):
        r = row0 + k * CROWS
        for a, b in zip(arrays, bufs):
            pltpu.make_async_copy(a.at[pl.ds(r, CROWS), :], b, sem).wait()

    start(bufs0, sem0, 0)

    def outer(k, carry):
        start(bufs1, sem1, 2 * k + 1)
        wait(bufs0, sem0, 2 * k)
        carry = compute(bufs0, carry)
        start(bufs0, sem0, 2 * k + 2)
        wait(bufs1, sem1, 2 * k + 1)
        carry = compute(bufs1, carry)
        return carry

    carry = lax.fori_loop(0, nchunk // 2 - 1, outer, carry)
    start(bufs1, sem1, nchunk - 1)
    wait(bufs0, sem0, nchunk - 2)
    carry = compute(bufs0, carry)
    wait(bufs1, sem1, nchunk - 1)
    carry = compute(bufs1, carry)
    return carry


def _group(buf, i):
    return buf[lax.shift_right_logical(i, 10), pl.ds(lax.bitwise_and(i, COLS - 1), L)]


def _pass1_body(nchunk, p_hbm, t_hbm, out_hbm, pb0, tb0, pb1, tb1, accb, sem0, sem1):
    def compute(bufs, acc):
        pb, tb = bufs

        def inner(j, acc):
            acc = list(acc)
            for u in range(UNROLL):
                o = j * (L * UNROLL) + u * L
                p = _group(pb, o)
                t = _group(tb, o)
                a_i, a_p, a_t = acc[u]
                acc[u] = (a_i + p * t, a_p + p, a_t + t)
            return tuple(acc)

        return lax.fori_loop(0, CHUNK // (L * UNROLL), inner, acc)

    z = jnp.zeros((L,), jnp.float32)
    acc = _stream_loop((p_hbm, t_hbm), (pb0, tb0), (pb1, tb1),
                       sem0, sem1, nchunk, compute, ((z, z, z),) * UNROLL)
    for i in range(3):
        v = acc[0][i]
        for u in range(1, UNROLL):
            v = v + acc[u][i]
        accb[i, :] = v
    accb[3, :] = jnp.zeros((L,), jnp.float32)
    pltpu.sync_copy(accb, out_hbm.at[_wid()])


def _tc1_body(p_ref, t_ref, o_ref):
    @pl.when(pl.program_id(0) == 0)
    def _init():
        o_ref[...] = jnp.zeros_like(o_ref)

    p = p_ref[...]
    t = t_ref[...]
    for arr, base in ((p * t, 0), (p, 8), (t, 16)):
        s = arr[0:8, :]
        for i in range(1, TCBR // 8):
            s = s + arr[8 * i:8 * (i + 1), :]
        o_ref[base:base + 8, :] += s


def _tc2_body(c_ref, p_ref, t_ref, w_ref, o_ref):
    @pl.when(pl.program_id(0) == 0)
    def _init():
        o_ref[...] = jnp.zeros_like(o_ref)

    c10 = c_ref[0]
    edges = [float(b) for b in range(BINS)] + [THRESH10]

    p = p_ref[...]
    t = t_ref[...]
    w = w_ref[...]
    g = jnp.abs(c10 * p - 10.0 * t)
    gm = jnp.where(w > 0.0, g, -1.0)
    pt = p * t
    for e_idx, e in enumerate(edges):
        mask = gm >= e
        cnt = mask.astype(jnp.float32)
        ptm = jnp.where(mask, pt, 0.0)
        a_c = cnt[0:8, :]
        a_p = ptm[0:8, :]
        for i in range(1, TCBR2 // 8):
            a_c = a_c + cnt[8 * i:8 * (i + 1), :]
            a_p = a_p + ptm[8 * i:8 * (i + 1), :]
        o_ref[e_idx, :, :] += a_c
        o_ref[NB + e_idx, :, :] += a_p


def _pass2_body(nchunk, p_hbm, t_hbm, w_hbm, c_hbm, out_hbm,
                pb0, tb0, wb0, pb1, tb1, wb1, cb, hist, sem0, sem1):
    pltpu.sync_copy(c_hbm, cb)
    c10 = cb[...]
    for i in range(2 * NB):
        hist[pl.ds(i * L, L)] = jnp.zeros((L,), jnp.float32)
    lane = lax.iota(jnp.int32, L)
    ones = jnp.ones((L,), jnp.float32)

    def compute(bufs, carry):
        pb, tb, wb = bufs

        @plsc.parallel_loop(0, CHUNK, L, unroll=8)
        def body(i):
            p = _group(pb, i)
            t = _group(tb, i)
            w = _group(wb, i)
            g10 = jnp.abs(c10 * p - 10.0 * t)
            b = jnp.where(g10 < THRESH10,
                          jnp.minimum(g10.astype(jnp.int32), BINS - 1), BINS)
            m = w > 0.0
            slot = b * L + lane
            plsc.addupdate_scatter(hist, [slot], ones, mask=m)
            plsc.addupdate_scatter(hist, [slot + NB * L], p * t, mask=m)

        return carry

    _stream_loop((p_hbm, t_hbm, w_hbm), (pb0, tb0, wb0), (pb1, tb1, wb1),
                 sem0, sem1, nchunk, compute, 0)
    pltpu.sync_copy(hist, out_hbm.at[_wid()])


@functools.cache
def _build(nrows):
    assert nrows % (NW * CROWS * 2) == 0, nrows
    nchunk1 = ROWS1_SC // (NW * CROWS)
    nchunk2 = ROWS2_SC // (NW * CROWS)
    assert nchunk1 % 2 == 0 and nchunk2 % 2 == 0
    rows_tc = nrows - ROWS1_SC
    row_off = ROWS1_SC // TCBR
    rows_tc2 = nrows - ROWS2_SC
    row_off2 = ROWS2_SC // TCBR2
    buf = lambda: pltpu.VMEM((CROWS, COLS), jnp.float32)
    params = pltpu.CompilerParams(needs_layout_passes=False)

    pass1 = pl.kernel(
        functools.partial(_pass1_body, nchunk1),
        out_type=jax.ShapeDtypeStruct((NW, 4, L), jnp.float32),
        mesh=_mesh(),
        compiler_params=params,
        scratch_types=[buf() for _ in range(4)]
        + [pltpu.VMEM((4, L), jnp.float32),
           pltpu.SemaphoreType.DMA, pltpu.SemaphoreType.DMA],
    )
    tc1 = pl.pallas_call(
        _tc1_body,
        grid=(rows_tc // TCBR,),
        in_specs=[pl.BlockSpec((TCBR, COLS), lambda k: (row_off + k, 0)),
                  pl.BlockSpec((TCBR, COLS), lambda k: (row_off + k, 0))],
        out_specs=pl.BlockSpec((24, COLS), lambda k: (0, 0)),
        out_shape=jax.ShapeDtypeStruct((24, COLS), jnp.float32),
    )
    tc2 = pl.pallas_call(
        _tc2_body,
        grid=(rows_tc2 // TCBR2,),
        in_specs=[pl.BlockSpec(memory_space=pltpu.SMEM),
                  pl.BlockSpec((TCBR2, COLS), lambda k: (row_off2 + k, 0)),
                  pl.BlockSpec((TCBR2, COLS), lambda k: (row_off2 + k, 0)),
                  pl.BlockSpec((TCBR2, COLS), lambda k: (row_off2 + k, 0))],
        out_specs=pl.BlockSpec((2 * NB, 8, COLS), lambda k: (0, 0, 0)),
        out_shape=jax.ShapeDtypeStruct((2 * NB, 8, COLS), jnp.float32),
    )
    pass2 = pl.kernel(
        functools.partial(_pass2_body, nchunk2),
        out_type=jax.ShapeDtypeStruct((NW, 2 * NB * L), jnp.float32),
        mesh=_mesh(),
        compiler_params=params,
        scratch_types=[buf() for _ in range(6)]
        + [pltpu.VMEM((L,), jnp.float32),
           pltpu.VMEM((2 * NB * L,), jnp.float32),
           pltpu.SemaphoreType.DMA, pltpu.SemaphoreType.DMA],
    )
    return pass1, tc1, pass2, tc2


def kernel(pred, target, label_weight):
    p = pred
    t = target.astype(jnp.float32)
    w = label_weight.astype(jnp.float32)
    pass1, tc1, pass2, tc2 = _build(p.shape[0])

    part1 = pass1(p, t)
    part1_tc = tc1(p, t)
    s = jnp.sum(part1, axis=(0, 2))
    i_sum = s[0] + jnp.sum(part1_tc[0:8])
    big_s = (s[1] + s[2]
             + jnp.sum(part1_tc[8:16]) + jnp.sum(part1_tc[16:24]))
    cvec = jnp.full((L,), 20.0 * i_sum / big_s, jnp.float32)

    part2 = pass2(p, t, w, cvec)
    part2_tc = tc2(cvec[:1], p, t, w)
    h = jnp.sum(part2.reshape(NW, 2 * NB, L), axis=(0, 2))
    cum_c = jnp.sum(part2_tc[:NB], axis=(1, 2))
    cum_p = jnp.sum(part2_tc[NB:], axis=(1, 2))
    zero1 = jnp.zeros((1,), jnp.float32)
    tc_counts = cum_c - jnp.concatenate([cum_c[1:], zero1])
    tc_pt = cum_p - jnp.concatenate([cum_p[1:], zero1])
    counts = h[:BINS] + tc_counts[:BINS]
    tot = jnp.maximum(jnp.sum(h[:NB]) + cum_c[0], 1.0)
    ptb = h[NB:NB + BINS] + tc_pt[:BINS]
    n = jnp.sum((counts > 0).astype(jnp.float32))
    contrib = jnp.where(counts > 0, tot / jnp.maximum(counts, 1.0), 0.0) * ptb
    loss = 1.0 - (2.0 * jnp.sum(contrib) / jnp.maximum(n, 1.0)) / big_s
    return loss


# speedup vs baseline: 1.0376x; 1.0095x over previous
import functools

import numpy as np
import jax
import jax.numpy as jnp
from jax import lax
from jax.experimental import pallas as pl
from jax.experimental.pallas import tpu as pltpu
from jax.experimental.pallas import tpu_sc as plsc

NC = 2
NS = 16
L = 16
NW = NC * NS
BINS = 10
NB = BINS + 1
COLS = 1024
CROWS = 16
CHUNK = CROWS * COLS
UNROLL = 4
TCBR = 512
ROWS1_SC = 8192
TCBR2 = 256
ROWS2_SC = 10240
THRESH10 = float(np.float32(10.0) * (np.float32(1.0) + np.float32(1e-6)))


def _wid():
    return lax.axis_index("s") * NC + lax.axis_index("c")


def _mesh():
    return plsc.VectorSubcoreMesh(
        core_axis_name="c", subcore_axis_name="s", num_cores=NC, num_subcores=NS
    )


def _stream_loop(arrays, bufs0, bufs1, sem0, sem1, nchunk, compute, carry):
    row0 = _wid() * (nchunk * CROWS)

    def start(bufs, sem, k):
        r = row0 + k * CROWS
        for a, b in zip(arrays, bufs):
            pltpu.async_copy(a.at[pl.ds(r, CROWS), :], b, sem)

    def wait(bufs, sem, k):
        r = row0 + k * CROWS
        for a, b in zip(arrays, bufs):
            pltpu.make_async_copy(a.at[pl.ds(r, CROWS), :], b, sem).wait()

    start(bufs0, sem0, 0)

    def outer(k, carry):
        start(bufs1, sem1, 2 * k + 1)
        wait(bufs0, sem0, 2 * k)
        carry = compute(bufs0, carry)
        start(bufs0, sem0, 2 * k + 2)
        wait(bufs1, sem1, 2 * k + 1)
        carry = compute(bufs1, carry)
        return carry

    carry = lax.fori_loop(0, nchunk // 2 - 1, outer, carry)
    start(bufs1, sem1, nchunk - 1)
    wait(bufs0, sem0, nchunk - 2)
    carry = compute(bufs0, carry)
    wait(bufs1, sem1, nchunk - 1)
    carry = compute(bufs1, carry)
    return carry


def _group(buf, i):
    return buf[lax.shift_right_logical(i, 10), pl.ds(lax.bitwise_and(i, COLS - 1), L)]


def _pass1_body(nchunk, p_hbm, t_hbm, out_hbm, pb0, tb0, pb1, tb1, accb, sem0, sem1):
    def compute(bufs, acc):
        pb, tb = bufs

        def inner(j, acc):
            acc = list(acc)
            for u in range(UNROLL):
                o = j * (L * UNROLL) + u * L
                p = _group(pb, o)
                t = _group(tb, o)
                a_i, a_p, a_t = acc[u]
                acc[u] = (a_i + p * t, a_p + p, a_t + t)
            return tuple(acc)

        return lax.fori_loop(0, CHUNK // (L * UNROLL), inner, acc)

    z = jnp.zeros((L,), jnp.float32)
    acc = _stream_loop((p_hbm, t_hbm), (pb0, tb0), (pb1, tb1),
                       sem0, sem1, nchunk, compute, ((z, z, z),) * UNROLL)
    for i in range(3):
        v = acc[0][i]
        for u in range(1, UNROLL):
            v = v + acc[u][i]
        accb[i, :] = v
    accb[3, :] = jnp.zeros((L,), jnp.float32)
    pltpu.sync_copy(accb, out_hbm.at[_wid()])


def _tc1_body(p_ref, t_ref, o_ref):
    @pl.when(pl.program_id(0) == 0)
    def _init():
        o_ref[...] = jnp.zeros_like(o_ref)

    p = p_ref[...]
    t = t_ref[...]
    for arr, base in ((p * t, 0), (p, 8), (t, 16)):
        s = arr[0:8, :]
        for i in range(1, TCBR // 8):
            s = s + arr[8 * i:8 * (i + 1), :]
        o_ref[base:base + 8, :] += s


def _tc2_body(c_ref, p_ref, t_ref, w_ref, o_ref):
    @pl.when(pl.program_id(0) == 0)
    def _init():
        o_ref[...] = jnp.zeros_like(o_ref)

    c10 = c_ref[0]
    edges = [float(b) for b in range(BINS)] + [THRESH10]

    p = p_ref[...]
    t = t_ref[...]
    w = w_ref[...]
    g = jnp.abs(c10 * p - 10.0 * t)
    gm = jnp.where(w > 0.0, g, -1.0)
    pt = p * t
    for e_idx, e in enumerate(edges):
        mask = gm >= e
        cnt = mask.astype(jnp.float32)
        ptm = jnp.where(mask, pt, 0.0)
        a_c = cnt[0:8, :]
        a_p = ptm[0:8, :]
        for i in range(1, TCBR2 // 8):
            a_c = a_c + cnt[8 * i:8 * (i + 1), :]
            a_p = a_p + ptm[8 * i:8 * (i + 1), :]
        o_ref[e_idx, :, :] += a_c
        o_ref[NB + e_idx, :, :] += a_p


def _pass2_body(nchunk, p_hbm, t_hbm, w_hbm, c_hbm, out_hbm,
                pb0, tb0, wb0, pb1, tb1, wb1, cb, hist, sem0, sem1):
    pltpu.sync_copy(c_hbm, cb)
    c10 = cb[...]
    for i in range(2 * NB):
        hist[pl.ds(i * L, L)] = jnp.zeros((L,), jnp.float32)
    lane = lax.iota(jnp.int32, L)
    ones = jnp.ones((L,), jnp.float32)

    def compute(bufs, carry):
        pb, tb, wb = bufs

        @plsc.parallel_loop(0, CHUNK, L, unroll=8)
        def body(i):
            p = _group(pb, i)
            t = _group(tb, i)
            w = _group(wb, i)
            g10 = jnp.abs(c10 * p - 10.0 * t)
            b = jnp.where(g10 < THRESH10,
                          jnp.minimum(g10.astype(jnp.int32), BINS - 1), BINS)
            m = w > 0.0
            slot = b * L + lane
            plsc.addupdate_scatter(hist, [slot], ones, mask=m)
            plsc.addupdate_scatter(hist, [slot + NB * L], p * t, mask=m)

        return carry

    _stream_loop((p_hbm, t_hbm, w_hbm), (pb0, tb0, wb0), (pb1, tb1, wb1),
                 sem0, sem1, nchunk, compute, 0)
    pltpu.sync_copy(hist, out_hbm.at[_wid()])


@functools.cache
def _build(nrows):
    assert nrows % (NW * CROWS * 2) == 0, nrows
    nchunk1 = ROWS1_SC // (NW * CROWS)
    nchunk2 = ROWS2_SC // (NW * CROWS)
    assert nchunk1 % 2 == 0 and nchunk2 % 2 == 0
    rows_tc = nrows - ROWS1_SC
    row_off = ROWS1_SC // TCBR
    rows_tc2 = nrows - ROWS2_SC
    row_off2 = ROWS2_SC // TCBR2
    buf = lambda: pltpu.VMEM((CROWS, COLS), jnp.float32)
    params = pltpu.CompilerParams(needs_layout_passes=False)

    pass1 = pl.kernel(
        functools.partial(_pass1_body, nchunk1),
        out_type=jax.ShapeDtypeStruct((NW, 4, L), jnp.float32),
        mesh=_mesh(),
        compiler_params=params,
        scratch_types=[buf() for _ in range(4)]
        + [pltpu.VMEM((4, L), jnp.float32),
           pltpu.SemaphoreType.DMA, pltpu.SemaphoreType.DMA],
    )
    tc1 = pl.pallas_call(
        _tc1_body,
        grid=(rows_tc // TCBR,),
        in_specs=[pl.BlockSpec((TCBR, COLS), lambda k: (row_off + k, 0)),
                  pl.BlockSpec((TCBR, COLS), lambda k: (row_off + k, 0))],
        out_specs=pl.BlockSpec((24, COLS), lambda k: (0, 0)),
        out_shape=jax.ShapeDtypeStruct((24, COLS), jnp.float32),
    )
    tc2 = pl.pallas_call(
        _tc2_body,
        grid=(rows_tc2 // TCBR2,),
        in_specs=[pl.BlockSpec(memory_space=pltpu.SMEM),
                  pl.BlockSpec((TCBR2, COLS), lambda k: (row_off2 + k, 0)),
                  pl.BlockSpec((TCBR2, COLS), lambda k: (row_off2 + k, 0)),
                  pl.BlockSpec((TCBR2, COLS), lambda k: (row_off2 + k, 0))],
        out_specs=pl.BlockSpec((2 * NB, 8, COLS), lambda k: (0, 0, 0)),
        out_shape=jax.ShapeDtypeStruct((2 * NB, 8, COLS), jnp.float32),
    )
    pass2 = pl.kernel(
        functools.partial(_pass2_body, nchunk2),
        out_type=jax.ShapeDtypeStruct((NW, 2 * NB * L), jnp.float32),
        mesh=_mesh(),
        compiler_params=params,
        scratch_types=[buf() for _ in range(6)]
        + [pltpu.VMEM((L,), jnp.float32),
           pltpu.VMEM((2 * NB * L,), jnp.float32),
           pltpu.SemaphoreType.DMA, pltpu.SemaphoreType.DMA],
    )
    return pass1, tc1, pass2, tc2


def kernel(pred, target, label_weight):
    p = pred
    t = target.astype(jnp.float32)
    w = label_weight.astype(jnp.float32)
    pass1, tc1, pass2, tc2 = _build(p.shape[0])

    part1 = pass1(p, t)
    part1_tc = tc1(p, t)
    s = jnp.sum(part1, axis=(0, 2))
    i_sum = s[0] + jnp.sum(part1_tc[0:8])
    big_s = (s[1] + s[2]
             + jnp.sum(part1_tc[8:16]) + jnp.sum(part1_tc[16:24]))
    cvec = jnp.full((L,), 20.0 * i_sum / big_s, jnp.float32)

    part2 = pass2(p, t, w, cvec)
    part2_tc = tc2(cvec[:1], p, t, w)
    h = jnp.sum(part2.reshape(NW, 2 * NB, L), axis=(0, 2))
    cum_c = jnp.sum(part2_tc[:NB], axis=(1, 2))
    cum_p = jnp.sum(part2_tc[NB:], axis=(1, 2))
    zero1 = jnp.zeros((1,), jnp.float32)
    tc_counts = cum_c - jnp.concatenate([cum_c[1:], zero1])
    tc_pt = cum_p - jnp.concatenate([cum_p[1:], zero1])
    counts = h[:BINS] + tc_counts[:BINS]
    tot = jnp.maximum(jnp.sum(h[:NB]) + cum_c[0], 1.0)
    ptb = h[NB:NB + BINS] + tc_pt[:BINS]
    n = jnp.sum((counts > 0).astype(jnp.float32))
    contrib = jnp.where(counts > 0, tot / jnp.maximum(counts, 1.0), 0.0) * ptb
    loss = 1.0 - (2.0 * jnp.sum(contrib) / jnp.maximum(n, 1.0)) / big_s
    return loss


# speedup vs baseline: 1.0431x; 1.0053x over previous
import functools

import numpy as np
import jax
import jax.numpy as jnp
from jax import lax
from jax.experimental import pallas as pl
from jax.experimental.pallas import tpu as pltpu
from jax.experimental.pallas import tpu_sc as plsc

NC = 2
NS = 16
L = 16
NW = NC * NS
BINS = 10
NB = BINS + 1
COLS = 1024
CROWS = 16
CHUNK = CROWS * COLS
UNROLL = 4
TCBR = 512
ROWS1_SC = 7168
TCBR2 = 256
ROWS2_SC = 10240
THRESH10 = float(np.float32(10.0) * (np.float32(1.0) + np.float32(1e-6)))


def _wid():
    return lax.axis_index("s") * NC + lax.axis_index("c")


def _mesh():
    return plsc.VectorSubcoreMesh(
        core_axis_name="c", subcore_axis_name="s", num_cores=NC, num_subcores=NS
    )


def _stream_loop(arrays, bufs0, bufs1, sem0, sem1, nchunk, compute, carry):
    row0 = _wid() * (nchunk * CROWS)

    def start(bufs, sem, k):
        r = row0 + k * CROWS
        for a, b in zip(arrays, bufs):
            pltpu.async_copy(a.at[pl.ds(r, CROWS), :], b, sem)

    def wait(bufs, sem, k):
        r = row0 + k * CROWS
        for a, b in zip(arrays, bufs):
            pltpu.make_async_copy(a.at[pl.ds(r, CROWS), :], b, sem).wait()

    start(bufs0, sem0, 0)

    def outer(k, carry):
        start(bufs1, sem1, 2 * k + 1)
        wait(bufs0, sem0, 2 * k)
        carry = compute(bufs0, carry)
        start(bufs0, sem0, 2 * k + 2)
        wait(bufs1, sem1, 2 * k + 1)
        carry = compute(bufs1, carry)
        return carry

    carry = lax.fori_loop(0, nchunk // 2 - 1, outer, carry)
    start(bufs1, sem1, nchunk - 1)
    wait(bufs0, sem0, nchunk - 2)
    carry = compute(bufs0, carry)
    wait(bufs1, sem1, nchunk - 1)
    carry = compute(bufs1, carry)
    return carry


def _group(buf, i):
    return buf[lax.shift_right_logical(i, 10), pl.ds(lax.bitwise_and(i, COLS - 1), L)]


def _pass1_body(nchunk, p_hbm, t_hbm, out_hbm, pb0, tb0, pb1, tb1, accb, sem0, sem1):
    def compute(bufs, acc):
        pb, tb = bufs

        def inner(j, acc):
            acc = list(acc)
            for u in range(UNROLL):
                o = j * (L * UNROLL) + u * L
                p = _group(pb, o)
                t = _group(tb, o)
                a_i, a_p, a_t = acc[u]
                acc[u] = (a_i + p * t, a_p + p, a_t + t)
            return tuple(acc)

        return lax.fori_loop(0, CHUNK // (L * UNROLL), inner, acc)

    z = jnp.zeros((L,), jnp.float32)
    acc = _stream_loop((p_hbm, t_hbm), (pb0, tb0), (pb1, tb1),
                       sem0, sem1, nchunk, compute, ((z, z, z),) * UNROLL)
    for i in range(3):
        v = acc[0][i]
        for u in range(1, UNROLL):
            v = v + acc[u][i]
        accb[i, :] = v
    accb[3, :] = jnp.zeros((L,), jnp.float32)
    pltpu.sync_copy(accb, out_hbm.at[_wid()])


def _tc1_body(p_ref, t_ref, o_ref):
    @pl.when(pl.program_id(0) == 0)
    def _init():
        o_ref[...] = jnp.zeros_like(o_ref)

    p = p_ref[...]
    t = t_ref[...]
    for arr, base in ((p * t, 0), (p, 8), (t, 16)):
        s = arr[0:8, :]
        for i in range(1, TCBR // 8):
            s = s + arr[8 * i:8 * (i + 1), :]
        o_ref[base:base + 8, :] += s


def _tc2_body(c_ref, p_ref, t_ref, w_ref, o_ref):
    @pl.when(pl.program_id(0) == 0)
    def _init():
        o_ref[...] = jnp.zeros_like(o_ref)

    c10 = c_ref[0]
    edges = [float(b) for b in range(BINS)] + [THRESH10]

    p = p_ref[...]
    t = t_ref[...]
    w = w_ref[...]
    g = jnp.abs(c10 * p - 10.0 * t)
    gm = jnp.where(w > 0.0, g, -1.0)
    pt = p * t
    for e_idx, e in enumerate(edges):
        mask = gm >= e
        cnt = mask.astype(jnp.float32)
        ptm = jnp.where(mask, pt, 0.0)
        a_c = cnt[0:8, :]
        a_p = ptm[0:8, :]
        for i in range(1, TCBR2 // 8):
            a_c = a_c + cnt[8 * i:8 * (i + 1), :]
            a_p = a_p + ptm[8 * i:8 * (i + 1), :]
        o_ref[e_idx, :, :] += a_c
        o_ref[NB + e_idx, :, :] += a_p


def _pass2_body(nchunk, p_hbm, t_hbm, w_hbm, c_hbm, out_hbm,
                pb0, tb0, wb0, pb1, tb1, wb1, cb, hist, sem0, sem1):
    pltpu.sync_copy(c_hbm, cb)
    c10 = cb[...]
    for i in range(2 * NB):
        hist[pl.ds(i * L, L)] = jnp.zeros((L,), jnp.float32)
    lane = lax.iota(jnp.int32, L)
    ones = jnp.ones((L,), jnp.float32)

    def compute(bufs, carry):
        pb, tb, wb = bufs

        @plsc.parallel_loop(0, CHUNK, L, unroll=8)
        def body(i):
            p = _group(pb, i)
            t = _group(tb, i)
            w = _group(wb, i)
            g10 = jnp.abs(c10 * p - 10.0 * t)
            b = jnp.where(g10 < THRESH10,
                          jnp.minimum(g10.astype(jnp.int32), BINS - 1), BINS)
            m = w > 0.0
            slot = b * L + lane
            plsc.addupdate_scatter(hist, [slot], ones, mask=m)
            plsc.addupdate_scatter(hist, [slot + NB * L], p * t, mask=m)

        return carry

    _stream_loop((p_hbm, t_hbm, w_hbm), (pb0, tb0, wb0), (pb1, tb1, wb1),
                 sem0, sem1, nchunk, compute, 0)
    pltpu.sync_copy(hist, out_hbm.at[_wid()])


@functools.cache
def _build(nrows):
    assert nrows % (NW * CROWS * 2) == 0, nrows
    nchunk1 = ROWS1_SC // (NW * CROWS)
    nchunk2 = ROWS2_SC // (NW * CROWS)
    assert nchunk1 % 2 == 0 and nchunk2 % 2 == 0
    rows_tc = nrows - ROWS1_SC
    row_off = ROWS1_SC // TCBR
    rows_tc2 = nrows - ROWS2_SC
    row_off2 = ROWS2_SC // TCBR2
    buf = lambda: pltpu.VMEM((CROWS, COLS), jnp.float32)
    params = pltpu.CompilerParams(needs_layout_passes=False)

    pass1 = pl.kernel(
        functools.partial(_pass1_body, nchunk1),
        out_type=jax.ShapeDtypeStruct((NW, 4, L), jnp.float32),
        mesh=_mesh(),
        compiler_params=params,
        scratch_types=[buf() for _ in range(4)]
        + [pltpu.VMEM((4, L), jnp.float32),
           pltpu.SemaphoreType.DMA, pltpu.SemaphoreType.DMA],
    )
    tc1 = pl.pallas_call(
        _tc1_body,
        grid=(rows_tc // TCBR,),
        in_specs=[pl.BlockSpec((TCBR, COLS), lambda k: (row_off + k, 0)),
                  pl.BlockSpec((TCBR, COLS), lambda k: (row_off + k, 0))],
        out_specs=pl.BlockSpec((24, COLS), lambda k: (0, 0)),
        out_shape=jax.ShapeDtypeStruct((24, COLS), jnp.float32),
    )
    tc2 = pl.pallas_call(
        _tc2_body,
        grid=(rows_tc2 // TCBR2,),
        in_specs=[pl.BlockSpec(memory_space=pltpu.SMEM),
                  pl.BlockSpec((TCBR2, COLS), lambda k: (row_off2 + k, 0)),
                  pl.BlockSpec((TCBR2, COLS), lambda k: (row_off2 + k, 0)),
                  pl.BlockSpec((TCBR2, COLS), lambda k: (row_off2 + k, 0))],
        out_specs=pl.BlockSpec((2 * NB, 8, COLS), lambda k: (0, 0, 0)),
        out_shape=jax.ShapeDtypeStruct((2 * NB, 8, COLS), jnp.float32),
    )
    pass2 = pl.kernel(
        functools.partial(_pass2_body, nchunk2),
        out_type=jax.ShapeDtypeStruct((NW, 2 * NB * L), jnp.float32),
        mesh=_mesh(),
        compiler_params=params,
        scratch_types=[buf() for _ in range(6)]
        + [pltpu.VMEM((L,), jnp.float32),
           pltpu.VMEM((2 * NB * L,), jnp.float32),
           pltpu.SemaphoreType.DMA, pltpu.SemaphoreType.DMA],
    )
    return pass1, tc1, pass2, tc2


def kernel(pred, target, label_weight):
    p = pred
    t = target.astype(jnp.float32)
    w = label_weight.astype(jnp.float32)
    pass1, tc1, pass2, tc2 = _build(p.shape[0])

    part1 = pass1(p, t)
    part1_tc = tc1(p, t)
    s = jnp.sum(part1, axis=(0, 2))
    i_sum = s[0] + jnp.sum(part1_tc[0:8])
    big_s = (s[1] + s[2]
             + jnp.sum(part1_tc[8:16]) + jnp.sum(part1_tc[16:24]))
    cvec = jnp.full((L,), 20.0 * i_sum / big_s, jnp.float32)

    part2 = pass2(p, t, w, cvec)
    part2_tc = tc2(cvec[:1], p, t, w)
    h = jnp.sum(part2.reshape(NW, 2 * NB, L), axis=(0, 2))
    cum_c = jnp.sum(part2_tc[:NB], axis=(1, 2))
    cum_p = jnp.sum(part2_tc[NB:], axis=(1, 2))
    zero1 = jnp.zeros((1,), jnp.float32)
    tc_counts = cum_c - jnp.concatenate([cum_c[1:], zero1])
    tc_pt = cum_p - jnp.concatenate([cum_p[1:], zero1])
    counts = h[:BINS] + tc_counts[:BINS]
    tot = jnp.maximum(jnp.sum(h[:NB]) + cum_c[0], 1.0)
    ptb = h[NB:NB + BINS] + tc_pt[:BINS]
    n = jnp.sum((counts > 0).astype(jnp.float32))
    contrib = jnp.where(counts > 0, tot / jnp.maximum(counts, 1.0), 0.0) * ptb
    loss = 1.0 - (2.0 * jnp.sum(contrib) / jnp.maximum(n, 1.0)) / big_s
    return loss
